# baseline (speedup 1.0000x reference)
import jax
import jax.numpy as jnp
from jax import lax
from jax.experimental import pallas as pl
from jax.experimental.pallas import tpu as pltpu
from jax.experimental.pallas import tpu_sc as plsc

B = 16384
EMB = 64
NCATE = 5
NC = 2
NS = 16
NW = NC * NS
BPW = B // NW
CH = 128
KCH = BPW // CH


def _din_body(uid_hbm, bid_hbm, cid_hbm, users_hbm, block_hbm, cat_hbm,
              out_u, out_b, out_c, uid_v, bid_v, cid_v, u_v, b_v, c_v,
              sem_g, sem_c):
    c = lax.axis_index("c")
    s = lax.axis_index("s")
    w = s * NC + c
    base = w * BPW

    pltpu.sync_copy(uid_hbm.at[w], uid_v)
    pltpu.sync_copy(bid_hbm.at[w], bid_v)
    pltpu.sync_copy(cid_hbm.at[w], cid_v)

    gath = []
    cat0 = []
    for k in range(KCH):
        rows = pl.ds(k * CH, CH)
        gath.append(pltpu.async_copy(
            users_hbm.at[uid_v.at[k]], u_v.at[rows], sem_g))
        gath.append(pltpu.async_copy(
            block_hbm.at[bid_v.at[k]], b_v.at[rows], sem_g))
        cat0.append(pltpu.async_copy(
            cat_hbm.at[cid_v.at[0, k]], c_v.at[rows], sem_c))
    for d in cat0:
        d.wait()
    catj = []
    for j in range(1, NCATE):
        for k in range(KCH):
            rows = pl.ds(k * CH, CH)
            catj.append(pltpu.async_copy(
                cat_hbm.at[cid_v.at[j, k]], c_v.at[rows], sem_c, add=True))
    for d in gath:
        d.wait()
    pltpu.sync_copy(u_v, out_u.at[pl.ds(base, BPW)])
    pltpu.sync_copy(b_v, out_b.at[pl.ds(base, BPW)])
    for d in catj:
        d.wait()
    pltpu.sync_copy(c_v, out_c.at[pl.ds(base, BPW)])


@jax.jit
def _din_sc(uid2, bid2, cid2, users_table, block_table, cat_scaled):
    mesh = plsc.VectorSubcoreMesh(core_axis_name="c", subcore_axis_name="s",
                                  num_cores=NC, num_subcores=NS)
    out_t = jax.ShapeDtypeStruct((B, EMB), jnp.float32)
    return pl.kernel(
        _din_body,
        out_type=(out_t, out_t, out_t),
        mesh=mesh,
        compiler_params=pltpu.CompilerParams(use_tc_tiling_on_sc=False),
        scratch_types=[
            pltpu.VMEM((KCH, CH), jnp.int32),
            pltpu.VMEM((KCH, CH), jnp.int32),
            pltpu.VMEM((NCATE, KCH, CH), jnp.int32),
            pltpu.VMEM((BPW, EMB), jnp.float32),
            pltpu.VMEM((BPW, EMB), jnp.float32),
            pltpu.VMEM((BPW, EMB), jnp.float32),
            pltpu.SemaphoreType.DMA,
            pltpu.SemaphoreType.DMA,
        ],
    )(uid2, bid2, cid2, users_table, block_table, cat_scaled)


def kernel(user_id, block_id, cate_idx, users_table, block_table,
           category_table):
    uid2 = user_id.astype(jnp.int32).reshape(NW, KCH, CH)
    bid2 = block_id.astype(jnp.int32).reshape(NW, KCH, CH)
    cid2 = (cate_idx.astype(jnp.int32).T.reshape(NCATE, NW, KCH, CH)
            .transpose(1, 0, 2, 3))
    cat_scaled = category_table * (1.0 / NCATE)
    u, b, c = _din_sc(uid2, bid2, cid2, users_table, block_table, cat_scaled)
    return jnp.concatenate([u, b, c], axis=1)
